# baseline (speedup 1.0000x reference)
import functools
import math

import jax
import jax.numpy as jnp
import numpy as np
from jax import lax
from jax.experimental import pallas as pl
from jax.experimental.pallas import tpu as pltpu
from jax.experimental.pallas import tpu_sc as plsc

_HID = 128
_HEADS = 8
_HC = 16
_LAYERS = 2
_K_SEQ = 32
_NN = {'gene': 18000, 'cpg': 40000, 'mirna': 2000}
_EREL = [
    ('cpg', 'regulates', 'gene'),
    ('gene', 'regulated_by', 'cpg'),
    ('mirna', 'targets', 'gene'),
    ('gene', 'targeted_by', 'mirna'),
    ('cpg', 'coregulates_cm', 'mirna'),
    ('mirna', 'coregulates_mc', 'cpg'),
    ('gene', 'ppi', 'gene'),
    ('gene', 'copathway', 'gene'),
    ('mirna', 'samefamily', 'mirna'),
]
_SELF = {'gene': 'self_loop_g', 'cpg': 'self_loop_c', 'mirna': 'self_loop_m'}

_NC, _NS, _NW = 2, 16, 32
_BK = 16
_CHMAX = 13440
_ROW = 144
_ZR = 848
_NDP = {'gene': 18048, 'cpg': 40064, 'mirna': 2048}
_BLKT = {'gene': 2256, 'cpg': 2504, 'mirna': 2048}


def _chunks(ndp):
    if ndp <= _CHMAX:
        return [(0, ndp)]
    nb = ndp // 128
    n = -(-ndp // _CHMAX)
    per = -(-nb // n)
    out, base = [], 0
    while base < nb:
        u = min(per, nb - base)
        out.append((base * 128, u * 128))
        base += u
    return out


_GDN = lax.GatherDimensionNumbers(
    offset_dims=(), collapsed_slice_dims=(0,), start_index_map=(0,))


def _lanesum(v):
    lanes = lax.iota(jnp.int32, 16)
    for k in (8, 4, 2, 1):
        idx = (lanes ^ k).reshape(16, 1)
        v = v + lax.gather(v, idx, _GDN, slice_sizes=(1,),
                           mode=lax.GatherScatterMode.PROMISE_IN_BOUNDS)
    return v


def _edge_stage(al, ar, src_pad, dst_pad, att, zeros, nd, nblk):
    chunks = _chunks(nd)
    mesh = plsc.VectorSubcoreMesh(core_axis_name="c", subcore_axis_name="s")

    @functools.partial(
        pl.kernel, mesh=mesh,
        out_type=jax.ShapeDtypeStruct((_NC, nd, _ROW), jnp.float32),
        compiler_params=pltpu.CompilerParams(use_tc_tiling_on_sc=False),
        scratch_types=[
            pltpu.VMEM((_BK,), jnp.int32),
            pltpu.VMEM((_BK,), jnp.int32),
            pltpu.VMEM((_BK,), jnp.int32),
            pltpu.VMEM((_BK, _HID), jnp.float32),
            pltpu.VMEM((_BK, _HID), jnp.float32),
            pltpu.VMEM((_BK, _ROW), jnp.float32),
            pltpu.VMEM((_HEADS, _HC), jnp.float32),
            pltpu.VMEM_SHARED((_CHMAX + 16, _ROW), jnp.float32),
            pltpu.SemaphoreType.DMA,
            pltpu.SemaphoreType.DMA,
        ])
    def ek(al_h, ar_h, src_h, dst_h, att_h, z_h, out_h,
           srcv, dstv, dlv, alr, arr, ctb, attv, shard, sem1, sem2):
        c = lax.axis_index("c")
        s = lax.axis_index("s")
        w = s * _NC + c
        pltpu.sync_copy(att_h, attv)
        pltpu.sync_copy(z_h.at[pl.ds(0, _BK)], ctb)
        lanes = lax.iota(jnp.int32, 16)
        for (base, rows) in chunks:
            rpt = rows // _NS
            pltpu.sync_copy(z_h.at[pl.ds(0, rpt)], shard.at[pl.ds(s * rpt, rpt)])
            plsc.subcore_barrier()

            def blk_body(b, carry):
                off = (w * nblk + b) * _BK
                pltpu.sync_copy(src_h.at[pl.ds(off, _BK)], srcv)
                pltpu.sync_copy(dst_h.at[pl.ds(off, _BK)], dstv)
                cp1 = pltpu.async_copy(al_h.at[srcv], alr, sem1)
                cp2 = pltpu.async_copy(ar_h.at[dstv], arr, sem2)
                for j in range(_BK // 16):
                    dd = dstv[pl.ds(j * 16, 16)]
                    inb = (dd >= base) & (dd < base + rows)
                    dlv[pl.ds(j * 16, 16)] = jnp.where(inb, dd - base,
                                                       _CHMAX + lanes)
                cp1.wait()
                cp2.wait()

                def edge_body(e, cc):
                    den = jnp.zeros((16,), jnp.float32)
                    for h in range(_HEADS):
                        av = alr[e, pl.ds(h * 16, 16)]
                        bv = arr[e, pl.ds(h * 16, 16)]
                        v = av + bv
                        m = jnp.maximum(v, 0.2 * v)
                        sc = jnp.minimum(_lanesum(m * attv[h]), 44.0)
                        ex = jnp.exp(sc)
                        ctb[e, pl.ds(h * 16, 16)] = ex * av
                        den = den + jnp.where(lanes == h, ex, 0.0)
                    ctb[e, pl.ds(_HID, 16)] = den
                    return cc
                lax.fori_loop(0, _BK, edge_body, 0, unroll=4)
                pltpu.sync_copy(ctb, shard.at[dlv], add=True)
                return carry
            lax.fori_loop(0, nblk, blk_body, 0)
            plsc.subcore_barrier()
            pltpu.sync_copy(shard.at[pl.ds(s * rpt, rpt)],
                            out_h.at[c, pl.ds(base + s * rpt, rpt)])
            plsc.subcore_barrier()

    return ek(al, ar, src_pad, dst_pad, att, zeros)


def _sc_gather(xc, xm, idxc, idxm):
    nr = idxc.shape[0]
    per = nr // _NW
    mesh = plsc.VectorSubcoreMesh(core_axis_name="c", subcore_axis_name="s")

    @functools.partial(
        pl.kernel, mesh=mesh,
        out_type=[jax.ShapeDtypeStruct((nr, _HID), jnp.float32),
                  jax.ShapeDtypeStruct((nr, _HID), jnp.float32)],
        scratch_types=[
            pltpu.VMEM((per,), jnp.int32),
            pltpu.VMEM((per, _HID), jnp.float32),
            pltpu.SemaphoreType.DMA,
        ])
    def gk(xc_h, xm_h, ic_h, im_h, oc_h, om_h, idxv, rows, sem):
        c = lax.axis_index("c")
        s = lax.axis_index("s")
        w = s * _NC + c
        base = w * per
        pltpu.sync_copy(ic_h.at[pl.ds(base, per)], idxv)
        pltpu.async_copy(xc_h.at[idxv], rows, sem).wait()
        pltpu.sync_copy(rows, oc_h.at[pl.ds(base, per)])
        pltpu.sync_copy(im_h.at[pl.ds(base, per)], idxv)
        pltpu.async_copy(xm_h.at[idxv], rows, sem).wait()
        pltpu.sync_copy(rows, om_h.at[pl.ds(base, per)])

    return gk(xc, xm, idxc, idxm)


def _project_multi(x, Ws, bs, blk=2000):
    n = x.shape[0]
    k = len(Ws)

    def body(x_ref, *refs):
        w_refs = refs[:k]
        b_refs = refs[k:2 * k]
        o_refs = refs[2 * k:]
        xv = x_ref[...]
        for j in range(k):
            o_refs[j][...] = (
                jnp.dot(xv, w_refs[j][...], preferred_element_type=jnp.float32)
                + b_refs[j][...])

    in_specs = ([pl.BlockSpec((blk, _HID), lambda i: (i, 0))]
                + [pl.BlockSpec((_HID, _HID), lambda i: (0, 0))] * k
                + [pl.BlockSpec((_HID,), lambda i: (0,))] * k)
    return pl.pallas_call(
        body,
        grid=(n // blk,),
        in_specs=in_specs,
        out_specs=[pl.BlockSpec((blk, _HID), lambda i: (i, 0))] * k,
        out_shape=[jax.ShapeDtypeStruct((n, _HID), jnp.float32)] * k,
    )(x, *Ws, *bs)


def _combine(x, wsl, bias_comb, parts, g, b, blk=2000):
    n = x.shape[0]
    nr = len(parts)
    expand = np.zeros((_HEADS, _HID), np.float32)
    for h in range(_HEADS):
        expand[h, h * 16:(h + 1) * 16] = 1.0
    expand = jnp.asarray(expand)

    def body(x_ref, w_ref, bc_ref, m_ref, g_ref, b_ref, *p_refs):
        o_ref = p_refs[-1]
        h = x_ref[...]
        acc = (jnp.dot(h, w_ref[...], preferred_element_type=jnp.float32)
               + bc_ref[...])
        for j in range(nr):
            pr = p_refs[2 * j][...] + p_refs[2 * j + 1][...]
            outu = pr[:, :_HID]
            den8 = pr[:, _HID:_HID + _HEADS]
            den = jnp.dot(den8, m_ref[...], preferred_element_type=jnp.float32)
            acc = acc + outu / (den + 1e-16)
        y = h + jnp.where(acc > 0, acc,
                          jnp.exp(jnp.minimum(acc, 0.0)) - 1.0)
        mu = jnp.mean(y, -1, keepdims=True)
        v = jnp.mean((y - mu) ** 2, -1, keepdims=True)
        o_ref[...] = (y - mu) / jnp.sqrt(v + 1e-5) * g_ref[...] + b_ref[...]

    in_specs = ([pl.BlockSpec((blk, _HID), lambda i: (i, 0)),
                 pl.BlockSpec((_HID, _HID), lambda i: (0, 0)),
                 pl.BlockSpec((_HID,), lambda i: (0,)),
                 pl.BlockSpec((_HEADS, _HID), lambda i: (0, 0)),
                 pl.BlockSpec((_HID,), lambda i: (0,)),
                 pl.BlockSpec((_HID,), lambda i: (0,))]
                + [pl.BlockSpec((blk, _ROW), lambda i: (i, 0))] * (2 * nr))
    flat_parts = []
    for (p0, p1) in parts:
        flat_parts += [p0, p1]
    return pl.pallas_call(
        body,
        grid=(n // blk,),
        in_specs=in_specs,
        out_specs=pl.BlockSpec((blk, _HID), lambda i: (i, 0)),
        out_shape=jax.ShapeDtypeStruct((n, _HID), jnp.float32),
    )(x, wsl, bias_comb, expand, g, b, *flat_parts)


def _zgene(ge, xg, g, b, nk):
    scale = 1.0 / math.sqrt(nk)

    def body(ge_ref, xg_ref, g_ref, b_ref, o_ref):
        y = jnp.dot(ge_ref[...], xg_ref[...],
                    preferred_element_type=jnp.float32) * scale
        mu = jnp.mean(y, -1, keepdims=True)
        v = jnp.mean((y - mu) ** 2, -1, keepdims=True)
        o_ref[...] = (y - mu) / jnp.sqrt(v + 1e-5) * g_ref[...] + b_ref[...]

    return pl.pallas_call(
        body,
        out_shape=jax.ShapeDtypeStruct((64, _HID), jnp.float32),
    )(ge, xg, g, b)


def _topk(X, blk=8):
    B, C = X.shape

    def body(x_ref, idx_ref, w_ref):
        x = x_ref[...]
        ax = jnp.abs(x)
        iot = lax.broadcasted_iota(jnp.int32, (blk, C), 1)
        for k in range(_K_SEQ):
            m = jnp.max(ax, axis=1, keepdims=True)
            sel = ax == m
            idx = jnp.min(jnp.where(sel, iot, C), axis=1, keepdims=True)
            hit = iot == idx
            idx_ref[:, k] = idx[:, 0]
            w_ref[:, k] = jnp.sum(jnp.where(hit, x, 0.0), axis=1)
            ax = jnp.where(hit, -1.0, ax)

    return pl.pallas_call(
        body,
        grid=(B // blk,),
        in_specs=[pl.BlockSpec((blk, C), lambda i: (i, 0))],
        out_specs=[pl.BlockSpec((blk, _K_SEQ), lambda i: (i, 0)),
                   pl.BlockSpec((blk, _K_SEQ), lambda i: (i, 0))],
        out_shape=[jax.ShapeDtypeStruct((B, _K_SEQ), jnp.int32),
                   jax.ShapeDtypeStruct((B, _K_SEQ), jnp.float32)],
    )(X)


def _seq_combine(Et, w, pos):
    B = w.shape[0]

    def body(e_ref, w_ref, p_ref, o_ref):
        o_ref[...] = (e_ref[...] * (1.0 + w_ref[...][..., None])
                      + p_ref[...][None])

    return pl.pallas_call(
        body,
        in_specs=[pl.BlockSpec((B, _K_SEQ, _HID), lambda: (0, 0, 0)),
                  pl.BlockSpec((B, _K_SEQ), lambda: (0, 0)),
                  pl.BlockSpec((_K_SEQ, _HID), lambda: (0, 0))],
        out_specs=pl.BlockSpec((B, _K_SEQ, _HID), lambda: (0, 0, 0)),
        out_shape=jax.ShapeDtypeStruct((B, _K_SEQ, _HID), jnp.float32),
    )(Et, w, pos)


def kernel(params, gene_expr, meth, mirna_expr, edges):
    epad = {}
    for (s, r, d) in _EREL:
        ei = edges[r]
        E = ei.shape[1]
        nblk = -(-E // (_NW * _BK))
        E_pad = nblk * _NW * _BK
        pad = E_pad - E
        src_p = jnp.concatenate([ei[0], jnp.zeros((pad,), jnp.int32)])
        dst_p = jnp.concatenate([ei[1], jnp.full((pad,), _NN[d], jnp.int32)])
        epad[r] = (src_p, dst_p, nblk)
    zeros = jnp.zeros((_ZR, _ROW), jnp.float32)

    x = {t: jnp.concatenate(
            [params['emb_' + t],
             jnp.zeros((_NDP[t] - _NN[t], _HID), jnp.float32)])
         for t in _NN}
    for i in range(_LAYERS):
        proj = {}
        for t in _NN:
            keys, Ws, bs = [], [], []
            for (s, r, d) in _EREL:
                p = params['conv%d_%s' % (i, r)]
                if s == t:
                    keys.append(('l', r)); Ws.append(p['Wl']); bs.append(p['bl'])
                if d == t:
                    keys.append(('r', r)); Ws.append(p['Wr']); bs.append(p['br'])
            outs = _project_multi(x[t], Ws, bs, blk=_BLKT[t])
            for kk, o in zip(keys, outs):
                proj[kk] = o
        partials = {}
        for (s, r, d) in _EREL:
            p = params['conv%d_%s' % (i, r)]
            src_p, dst_p, nblk = epad[r]
            P = _edge_stage(proj[('l', r)], proj[('r', r)], src_p, dst_p,
                            p['att'], zeros, _NDP[d], nblk)
            partials[r] = (P[0], P[1])
        xn = {}
        for t in _NN:
            psl = params['conv%d_%s' % (i, _SELF[t])]
            bias_comb = psl['bl'] + psl['bias']
            parts = []
            for (s, r, d) in _EREL:
                if d == t:
                    bias_comb = bias_comb + params['conv%d_%s' % (i, r)]['bias']
                    parts.append(partials[r])
            xn[t] = _combine(x[t], psl['Wl'], bias_comb, parts,
                             params['ln%d_%s_g' % (i, t)],
                             params['ln%d_%s_b' % (i, t)], blk=_BLKT[t])
        x = xn

    ge_pad = jnp.concatenate(
        [gene_expr, jnp.zeros((64, _NDP['gene'] - _NN['gene']), jnp.float32)],
        axis=1)
    z_gene = _zgene(ge_pad, x['gene'], params['gn_g'], params['gn_b'],
                    _NN['gene'])
    idxc, wc = _topk(meth)
    idxm, wm = _topk(mirna_expr)
    etc, etm = _sc_gather(x['cpg'], x['mirna'],
                          idxc.reshape(-1), idxm.reshape(-1))
    z_cpg = _seq_combine(etc.reshape(64, _K_SEQ, _HID), wc, params['cpg_pos'])
    z_mirna = _seq_combine(etm.reshape(64, _K_SEQ, _HID), wm,
                           params['mirna_pos'])
    return (z_gene, z_cpg, z_mirna)


# speedup vs baseline: 1.0690x; 1.0690x over previous
import functools
import math

import jax
import jax.numpy as jnp
import numpy as np
from jax import lax
from jax.experimental import pallas as pl
from jax.experimental.pallas import tpu as pltpu
from jax.experimental.pallas import tpu_sc as plsc

_HID = 128
_HEADS = 8
_HC = 16
_LAYERS = 2
_K_SEQ = 32
_NN = {'gene': 18000, 'cpg': 40000, 'mirna': 2000}
_EREL = [
    ('cpg', 'regulates', 'gene'),
    ('gene', 'regulated_by', 'cpg'),
    ('mirna', 'targets', 'gene'),
    ('gene', 'targeted_by', 'mirna'),
    ('cpg', 'coregulates_cm', 'mirna'),
    ('mirna', 'coregulates_mc', 'cpg'),
    ('gene', 'ppi', 'gene'),
    ('gene', 'copathway', 'gene'),
    ('mirna', 'samefamily', 'mirna'),
]
_SELF = {'gene': 'self_loop_g', 'cpg': 'self_loop_c', 'mirna': 'self_loop_m'}

_NC, _NS, _NW = 2, 16, 32
_BK = 64
_CHMAX = 10624
_ROW = 144
_ZR = 640
_NDP = {'gene': 18048, 'cpg': 40064, 'mirna': 2048}
_BLKT = {'gene': 2256, 'cpg': 2504, 'mirna': 2048}


def _chunks(ndp):
    if ndp <= _CHMAX:
        return [(0, ndp)]
    nb = ndp // 128
    n = -(-ndp // _CHMAX)
    per = -(-nb // n)
    out, base = [], 0
    while base < nb:
        u = min(per, nb - base)
        out.append((base * 128, u * 128))
        base += u
    return out


_GDN = lax.GatherDimensionNumbers(
    offset_dims=(), collapsed_slice_dims=(0,), start_index_map=(0,))


def _lanesum(v):
    lanes = lax.iota(jnp.int32, 16)
    for k in (8, 4, 2, 1):
        idx = (lanes ^ k).reshape(16, 1)
        v = v + lax.gather(v, idx, _GDN, slice_sizes=(1,),
                           mode=lax.GatherScatterMode.PROMISE_IN_BOUNDS)
    return v


def _edge_stage(al, ar, src_pad, dst_pad, att, zeros, nd, nblk):
    chunks = _chunks(nd)
    mesh = plsc.VectorSubcoreMesh(core_axis_name="c", subcore_axis_name="s")

    @functools.partial(
        pl.kernel, mesh=mesh,
        out_type=jax.ShapeDtypeStruct((_NC, nd, _ROW), jnp.float32),
        compiler_params=pltpu.CompilerParams(use_tc_tiling_on_sc=False),
        scratch_types=[
            pltpu.VMEM((_BK,), jnp.int32),
            pltpu.VMEM((_BK,), jnp.int32),
            pltpu.VMEM((_BK,), jnp.int32),
            pltpu.VMEM((_BK, _HID), jnp.float32),
            pltpu.VMEM((_BK, _HID), jnp.float32),
            pltpu.VMEM((_BK, _ROW), jnp.float32),
            pltpu.VMEM((_HEADS, _HC), jnp.float32),
            pltpu.VMEM_SHARED((_CHMAX + 16, _ROW), jnp.float32),
            pltpu.SemaphoreType.DMA,
            pltpu.SemaphoreType.DMA,
        ])
    def ek(al_h, ar_h, src_h, dst_h, att_h, z_h, out_h,
           srcv, dstv, dlv, alr, arr, ctb, attv, shard, sem1, sem2):
        c = lax.axis_index("c")
        s = lax.axis_index("s")
        w = s * _NC + c
        pltpu.sync_copy(att_h, attv)
        pltpu.sync_copy(z_h.at[pl.ds(0, _BK)], ctb)
        lanes = lax.iota(jnp.int32, 16)
        for (base, rows) in chunks:
            rpt = rows // _NS
            pltpu.sync_copy(z_h.at[pl.ds(0, rpt)], shard.at[pl.ds(s * rpt, rpt)])
            plsc.subcore_barrier()

            def blk_body(b, carry):
                off = (w * nblk + b) * _BK
                pltpu.sync_copy(src_h.at[pl.ds(off, _BK)], srcv)
                pltpu.sync_copy(dst_h.at[pl.ds(off, _BK)], dstv)
                cp1 = pltpu.async_copy(al_h.at[srcv], alr, sem1)
                cp2 = pltpu.async_copy(ar_h.at[dstv], arr, sem2)
                for j in range(_BK // 16):
                    dd = dstv[pl.ds(j * 16, 16)]
                    inb = (dd >= base) & (dd < base + rows)
                    dlv[pl.ds(j * 16, 16)] = jnp.where(inb, dd - base,
                                                       _CHMAX + lanes)
                cp1.wait()
                cp2.wait()

                def edge_body(e, cc):
                    den = jnp.zeros((16,), jnp.float32)
                    for h in range(_HEADS):
                        av = alr[e, pl.ds(h * 16, 16)]
                        bv = arr[e, pl.ds(h * 16, 16)]
                        v = av + bv
                        m = jnp.maximum(v, 0.2 * v)
                        sc = jnp.minimum(_lanesum(m * attv[h]), 44.0)
                        ex = jnp.exp(sc)
                        ctb[e, pl.ds(h * 16, 16)] = ex * av
                        den = den + jnp.where(lanes == h, ex, 0.0)
                    ctb[e, pl.ds(_HID, 16)] = den
                    return cc
                lax.fori_loop(0, _BK, edge_body, 0, unroll=4)
                pltpu.sync_copy(ctb, shard.at[dlv], add=True)
                return carry
            lax.fori_loop(0, nblk, blk_body, 0)
            plsc.subcore_barrier()
            pltpu.sync_copy(shard.at[pl.ds(s * rpt, rpt)],
                            out_h.at[c, pl.ds(base + s * rpt, rpt)])
            plsc.subcore_barrier()

    return ek(al, ar, src_pad, dst_pad, att, zeros)


def _sc_gather(xc, xm, idxc, idxm):
    nr = idxc.shape[0]
    per = nr // _NW
    mesh = plsc.VectorSubcoreMesh(core_axis_name="c", subcore_axis_name="s")

    @functools.partial(
        pl.kernel, mesh=mesh,
        out_type=[jax.ShapeDtypeStruct((nr, _HID), jnp.float32),
                  jax.ShapeDtypeStruct((nr, _HID), jnp.float32)],
        scratch_types=[
            pltpu.VMEM((per,), jnp.int32),
            pltpu.VMEM((per, _HID), jnp.float32),
            pltpu.SemaphoreType.DMA,
        ])
    def gk(xc_h, xm_h, ic_h, im_h, oc_h, om_h, idxv, rows, sem):
        c = lax.axis_index("c")
        s = lax.axis_index("s")
        w = s * _NC + c
        base = w * per
        pltpu.sync_copy(ic_h.at[pl.ds(base, per)], idxv)
        pltpu.async_copy(xc_h.at[idxv], rows, sem).wait()
        pltpu.sync_copy(rows, oc_h.at[pl.ds(base, per)])
        pltpu.sync_copy(im_h.at[pl.ds(base, per)], idxv)
        pltpu.async_copy(xm_h.at[idxv], rows, sem).wait()
        pltpu.sync_copy(rows, om_h.at[pl.ds(base, per)])

    return gk(xc, xm, idxc, idxm)


def _project_multi(x, Ws, bs, blk=2000):
    n = x.shape[0]
    k = len(Ws)

    def body(x_ref, *refs):
        w_refs = refs[:k]
        b_refs = refs[k:2 * k]
        o_refs = refs[2 * k:]
        xv = x_ref[...]
        for j in range(k):
            o_refs[j][...] = (
                jnp.dot(xv, w_refs[j][...], preferred_element_type=jnp.float32)
                + b_refs[j][...])

    in_specs = ([pl.BlockSpec((blk, _HID), lambda i: (i, 0))]
                + [pl.BlockSpec((_HID, _HID), lambda i: (0, 0))] * k
                + [pl.BlockSpec((_HID,), lambda i: (0,))] * k)
    return pl.pallas_call(
        body,
        grid=(n // blk,),
        in_specs=in_specs,
        out_specs=[pl.BlockSpec((blk, _HID), lambda i: (i, 0))] * k,
        out_shape=[jax.ShapeDtypeStruct((n, _HID), jnp.float32)] * k,
    )(x, *Ws, *bs)


def _combine(x, wsl, bias_comb, parts, g, b, blk=2000):
    n = x.shape[0]
    nr = len(parts)
    expand = np.zeros((_HEADS, _HID), np.float32)
    for h in range(_HEADS):
        expand[h, h * 16:(h + 1) * 16] = 1.0
    expand = jnp.asarray(expand)

    def body(x_ref, w_ref, bc_ref, m_ref, g_ref, b_ref, *p_refs):
        o_ref = p_refs[-1]
        h = x_ref[...]
        acc = (jnp.dot(h, w_ref[...], preferred_element_type=jnp.float32)
               + bc_ref[...])
        for j in range(nr):
            pr = p_refs[2 * j][...] + p_refs[2 * j + 1][...]
            outu = pr[:, :_HID]
            den8 = pr[:, _HID:_HID + _HEADS]
            den = jnp.dot(den8, m_ref[...], preferred_element_type=jnp.float32)
            acc = acc + outu / (den + 1e-16)
        y = h + jnp.where(acc > 0, acc,
                          jnp.exp(jnp.minimum(acc, 0.0)) - 1.0)
        mu = jnp.mean(y, -1, keepdims=True)
        v = jnp.mean((y - mu) ** 2, -1, keepdims=True)
        o_ref[...] = (y - mu) / jnp.sqrt(v + 1e-5) * g_ref[...] + b_ref[...]

    in_specs = ([pl.BlockSpec((blk, _HID), lambda i: (i, 0)),
                 pl.BlockSpec((_HID, _HID), lambda i: (0, 0)),
                 pl.BlockSpec((_HID,), lambda i: (0,)),
                 pl.BlockSpec((_HEADS, _HID), lambda i: (0, 0)),
                 pl.BlockSpec((_HID,), lambda i: (0,)),
                 pl.BlockSpec((_HID,), lambda i: (0,))]
                + [pl.BlockSpec((blk, _ROW), lambda i: (i, 0))] * (2 * nr))
    flat_parts = []
    for (p0, p1) in parts:
        flat_parts += [p0, p1]
    return pl.pallas_call(
        body,
        grid=(n // blk,),
        in_specs=in_specs,
        out_specs=pl.BlockSpec((blk, _HID), lambda i: (i, 0)),
        out_shape=jax.ShapeDtypeStruct((n, _HID), jnp.float32),
    )(x, wsl, bias_comb, expand, g, b, *flat_parts)


def _zgene(ge, xg, g, b, nk):
    scale = 1.0 / math.sqrt(nk)

    def body(ge_ref, xg_ref, g_ref, b_ref, o_ref):
        y = jnp.dot(ge_ref[...], xg_ref[...],
                    preferred_element_type=jnp.float32) * scale
        mu = jnp.mean(y, -1, keepdims=True)
        v = jnp.mean((y - mu) ** 2, -1, keepdims=True)
        o_ref[...] = (y - mu) / jnp.sqrt(v + 1e-5) * g_ref[...] + b_ref[...]

    return pl.pallas_call(
        body,
        out_shape=jax.ShapeDtypeStruct((64, _HID), jnp.float32),
    )(ge, xg, g, b)


def _topk(X, blk=8):
    B, C = X.shape

    def body(x_ref, idx_ref, w_ref):
        x = x_ref[...]
        ax = jnp.abs(x)
        iot = lax.broadcasted_iota(jnp.int32, (blk, C), 1)
        for k in range(_K_SEQ):
            m = jnp.max(ax, axis=1, keepdims=True)
            sel = ax == m
            idx = jnp.min(jnp.where(sel, iot, C), axis=1, keepdims=True)
            hit = iot == idx
            idx_ref[:, k] = idx[:, 0]
            w_ref[:, k] = jnp.sum(jnp.where(hit, x, 0.0), axis=1)
            ax = jnp.where(hit, -1.0, ax)

    return pl.pallas_call(
        body,
        grid=(B // blk,),
        in_specs=[pl.BlockSpec((blk, C), lambda i: (i, 0))],
        out_specs=[pl.BlockSpec((blk, _K_SEQ), lambda i: (i, 0)),
                   pl.BlockSpec((blk, _K_SEQ), lambda i: (i, 0))],
        out_shape=[jax.ShapeDtypeStruct((B, _K_SEQ), jnp.int32),
                   jax.ShapeDtypeStruct((B, _K_SEQ), jnp.float32)],
    )(X)


def _seq_combine(Et, w, pos):
    B = w.shape[0]

    def body(e_ref, w_ref, p_ref, o_ref):
        o_ref[...] = (e_ref[...] * (1.0 + w_ref[...][..., None])
                      + p_ref[...][None])

    return pl.pallas_call(
        body,
        in_specs=[pl.BlockSpec((B, _K_SEQ, _HID), lambda: (0, 0, 0)),
                  pl.BlockSpec((B, _K_SEQ), lambda: (0, 0)),
                  pl.BlockSpec((_K_SEQ, _HID), lambda: (0, 0))],
        out_specs=pl.BlockSpec((B, _K_SEQ, _HID), lambda: (0, 0, 0)),
        out_shape=jax.ShapeDtypeStruct((B, _K_SEQ, _HID), jnp.float32),
    )(Et, w, pos)


def kernel(params, gene_expr, meth, mirna_expr, edges):
    epad = {}
    for (s, r, d) in _EREL:
        ei = edges[r]
        E = ei.shape[1]
        nblk = -(-E // (_NW * _BK))
        E_pad = nblk * _NW * _BK
        pad = E_pad - E
        src_p = jnp.concatenate([ei[0], jnp.zeros((pad,), jnp.int32)])
        dst_p = jnp.concatenate([ei[1], jnp.full((pad,), _NN[d], jnp.int32)])
        epad[r] = (src_p, dst_p, nblk)
    zeros = jnp.zeros((_ZR, _ROW), jnp.float32)

    x = {t: jnp.concatenate(
            [params['emb_' + t],
             jnp.zeros((_NDP[t] - _NN[t], _HID), jnp.float32)])
         for t in _NN}
    for i in range(_LAYERS):
        proj = {}
        for t in _NN:
            keys, Ws, bs = [], [], []
            for (s, r, d) in _EREL:
                p = params['conv%d_%s' % (i, r)]
                if s == t:
                    keys.append(('l', r)); Ws.append(p['Wl']); bs.append(p['bl'])
                if d == t:
                    keys.append(('r', r)); Ws.append(p['Wr']); bs.append(p['br'])
            outs = _project_multi(x[t], Ws, bs, blk=_BLKT[t])
            for kk, o in zip(keys, outs):
                proj[kk] = o
        partials = {}
        for (s, r, d) in _EREL:
            p = params['conv%d_%s' % (i, r)]
            src_p, dst_p, nblk = epad[r]
            P = _edge_stage(proj[('l', r)], proj[('r', r)], src_p, dst_p,
                            p['att'], zeros, _NDP[d], nblk)
            partials[r] = (P[0], P[1])
        xn = {}
        for t in _NN:
            psl = params['conv%d_%s' % (i, _SELF[t])]
            bias_comb = psl['bl'] + psl['bias']
            parts = []
            for (s, r, d) in _EREL:
                if d == t:
                    bias_comb = bias_comb + params['conv%d_%s' % (i, r)]['bias']
                    parts.append(partials[r])
            xn[t] = _combine(x[t], psl['Wl'], bias_comb, parts,
                             params['ln%d_%s_g' % (i, t)],
                             params['ln%d_%s_b' % (i, t)], blk=_BLKT[t])
        x = xn

    ge_pad = jnp.concatenate(
        [gene_expr, jnp.zeros((64, _NDP['gene'] - _NN['gene']), jnp.float32)],
        axis=1)
    z_gene = _zgene(ge_pad, x['gene'], params['gn_g'], params['gn_b'],
                    _NN['gene'])
    idxc, wc = _topk(meth)
    idxm, wm = _topk(mirna_expr)
    etc, etm = _sc_gather(x['cpg'], x['mirna'],
                          idxc.reshape(-1), idxm.reshape(-1))
    z_cpg = _seq_combine(etc.reshape(64, _K_SEQ, _HID), wc, params['cpg_pos'])
    z_mirna = _seq_combine(etm.reshape(64, _K_SEQ, _HID), wm,
                           params['mirna_pos'])
    return (z_gene, z_cpg, z_mirna)
